# baseline (speedup 1.0000x reference)
import functools

import jax
import jax.numpy as jnp
from jax import lax
from jax.experimental import pallas as pl
from jax.experimental.pallas import tpu as pltpu
from jax.experimental.pallas import tpu_sc as plsc

K = 512
D = 32
BLK = 1024
N = 65536
NB = N // BLK

_NC, _NS = 2, 16
NW = _NC * _NS
BPW = N // NW
CHUNK = 128
NCHUNK = BPW // CHUNK


def _tc_body(x_ref, cb_ref, idx_ref, stats_ref, counts_ref):
    step = pl.program_id(0)

    @pl.when(step == 0)
    def _init():
        counts_ref[...] = jnp.zeros_like(counts_ref)

    xb = x_ref[...]
    cb = cb_ref[...]
    csq = jnp.sum(cb * cb, axis=1, keepdims=True)
    prod = lax.dot_general(cb, xb, (((1,), (1,)), ((), ())),
                           preferred_element_type=jnp.float32)
    d = csq - 2.0 * prod
    dmin = jnp.min(d, axis=0, keepdims=True)
    iota_k = lax.broadcasted_iota(jnp.int32, (K, BLK), 0)
    idx2 = jnp.min(jnp.where(d <= dmin, iota_k, K), axis=0,
                   keepdims=True)
    idx_ref[0, 0, :] = idx2[0]
    onehot = (iota_k == idx2).astype(jnp.float32)
    counts_ref[...] += jnp.sum(onehot, axis=1, keepdims=True)

    @pl.when(step == NB - 1)
    def _finish():
        p = counts_ref[...] * (1.0 / N)
        ent = -jnp.sum(p * jnp.log(p + 1e-10))
        perp = jnp.exp(ent)
        div = 1.0 - ent / jnp.log(jnp.float32(K))
        lane = lax.broadcasted_iota(jnp.int32, (1, 128), 1)
        stats_ref[...] = jnp.where(lane == 0, perp,
                                   jnp.where(lane == 1, div, 0.0))


_tc_call = pl.pallas_call(
    _tc_body,
    grid=(NB,),
    in_specs=[
        pl.BlockSpec((BLK, D), lambda i: (i, 0)),
        pl.BlockSpec((K, D), lambda i: (0, 0)),
    ],
    out_specs=[
        pl.BlockSpec((1, 1, BLK), lambda i: (i, 0, 0)),
        pl.BlockSpec((1, 128), lambda i: (0, 0)),
    ],
    out_shape=[
        jax.ShapeDtypeStruct((NB, 1, BLK), jnp.int32),
        jax.ShapeDtypeStruct((1, 128), jnp.float32),
    ],
    scratch_shapes=[pltpu.VMEM((K, 1), jnp.float32)],
)


@functools.cache
def _make_sc_gather():
    @functools.partial(
        pl.kernel,
        mesh=plsc.VectorSubcoreMesh(core_axis_name="c", subcore_axis_name="s"),
        compiler_params=pltpu.CompilerParams(use_tc_tiling_on_sc=False),
        out_type=jax.ShapeDtypeStruct((N, D), jnp.float32),
        scratch_types=[
            pltpu.VMEM((NCHUNK, CHUNK), jnp.int32),
            pltpu.VMEM((BPW, D), jnp.float32),
            pltpu.SemaphoreType.DMA,
        ],
    )
    def _sc_gather(cb_hbm, idx_hbm, out_hbm, idx_v, rows_v, sem):
        wid = lax.axis_index("s") * _NC + lax.axis_index("c")
        base = wid * BPW
        pltpu.sync_copy(idx_hbm.at[wid], idx_v)
        copies = []
        for j in range(NCHUNK):
            copies.append(
                pltpu.async_copy(cb_hbm.at[idx_v.at[j]],
                                 rows_v.at[pl.ds(j * CHUNK, CHUNK)], sem))
        for c in copies:
            c.wait()
        pltpu.sync_copy(rows_v, out_hbm.at[pl.ds(base, BPW)])

    return _sc_gather


def kernel(x, codebook):
    xr = x.reshape(N, D)
    idx3, stats = _tc_call(xr, codebook)
    idx = idx3.reshape(N)
    quantized = _make_sc_gather()(codebook, idx.reshape(NW, NCHUNK, CHUNK))
    return quantized, idx, stats[0, 0], stats[0, 1]


# speedup vs baseline: 1.0492x; 1.0492x over previous
import functools

import jax
import jax.numpy as jnp
from jax import lax
from jax.experimental import pallas as pl
from jax.experimental.pallas import tpu as pltpu
from jax.experimental.pallas import tpu_sc as plsc

K = 512
D = 32
BLK = 16384
N = 65536
NB = N // BLK

_NC, _NS = 2, 16
NW = _NC * _NS
BPW = N // NW
CHUNK = 128
NCHUNK = BPW // CHUNK


def _tc_body(x_ref, cb_ref, idx_ref, stats_ref, counts_ref, hcsq_ref):
    step = pl.program_id(0)

    @pl.when(step == 0)
    def _init():
        counts_ref[...] = jnp.zeros_like(counts_ref)
        cb0 = cb_ref[...]
        hcsq_ref[...] = 0.5 * jnp.sum(cb0 * cb0, axis=1, keepdims=True)

    xb = x_ref[...]
    prod = lax.dot_general(cb_ref[...], xb, (((1,), (1,)), ((), ())),
                           preferred_element_type=jnp.float32)
    h = prod - hcsq_ref[...]
    hmax = jnp.max(h, axis=0, keepdims=True)
    mask = h >= hmax
    riota = (K - 1) - lax.broadcasted_iota(jnp.int32, (K, BLK), 0)
    rm = jnp.max(jnp.where(mask, riota, 0), axis=0, keepdims=True)
    idx_ref[0, 0, :] = (K - 1) - rm[0]
    counts_ref[...] += jnp.sum(jnp.where(mask, 1, 0), axis=1, keepdims=True)

    @pl.when(step == NB - 1)
    def _finish():
        p = counts_ref[...].astype(jnp.float32) * (1.0 / N)
        ent = -jnp.sum(p * jnp.log(p + 1e-10))
        perp = jnp.exp(ent)
        div = 1.0 - ent / jnp.log(jnp.float32(K))
        lane = lax.broadcasted_iota(jnp.int32, (1, 128), 1)
        stats_ref[...] = jnp.where(lane == 0, perp,
                                   jnp.where(lane == 1, div, 0.0))


_tc_call = pl.pallas_call(
    _tc_body,
    grid=(NB,),
    in_specs=[
        pl.BlockSpec((BLK, D), lambda i: (i, 0)),
        pl.BlockSpec((K, D), lambda i: (0, 0)),
    ],
    out_specs=[
        pl.BlockSpec((1, 1, BLK), lambda i: (i, 0, 0)),
        pl.BlockSpec((1, 128), lambda i: (0, 0)),
    ],
    out_shape=[
        jax.ShapeDtypeStruct((NB, 1, BLK), jnp.int32),
        jax.ShapeDtypeStruct((1, 128), jnp.float32),
    ],
    scratch_shapes=[pltpu.VMEM((K, 1), jnp.int32),
                    pltpu.VMEM((K, 1), jnp.float32)],
)


@functools.cache
def _make_sc_gather():
    @functools.partial(
        pl.kernel,
        mesh=plsc.VectorSubcoreMesh(core_axis_name="c", subcore_axis_name="s"),
        compiler_params=pltpu.CompilerParams(use_tc_tiling_on_sc=False),
        out_type=jax.ShapeDtypeStruct((N, D), jnp.float32),
        scratch_types=[
            pltpu.VMEM((BPW,), jnp.int32),
            pltpu.VMEM((BPW, D), jnp.float32),
            pltpu.SemaphoreType.DMA,
            pltpu.SemaphoreType.DMA,
        ],
    )
    def _sc_gather(cb_hbm, idx_hbm, out_hbm, idx_v, rows_v, gsem, osem):
        wid = lax.axis_index("s") * _NC + lax.axis_index("c")
        base = wid * BPW
        pltpu.sync_copy(idx_hbm.at[pl.ds(base, BPW)], idx_v)
        gathers = []
        outs = []
        for j in range(NCHUNK):
            gathers.append(
                pltpu.async_copy(cb_hbm.at[idx_v.at[pl.ds(j * CHUNK, CHUNK)]],
                                 rows_v.at[pl.ds(j * CHUNK, CHUNK)], gsem))
        for j in range(NCHUNK):
            gathers[j].wait()
            outs.append(
                pltpu.async_copy(rows_v.at[pl.ds(j * CHUNK, CHUNK)],
                                 out_hbm.at[pl.ds(base + j * CHUNK, CHUNK)],
                                 osem))
        for c in outs:
            c.wait()

    return _sc_gather


def kernel(x, codebook):
    xr = x.reshape(N, D)
    idx3, stats = _tc_call(xr, codebook)
    idx = idx3.reshape(N)
    quantized = _make_sc_gather()(codebook, idx)
    return quantized, idx, stats[0, 0], stats[0, 1]


# speedup vs baseline: 1.0890x; 1.0380x over previous
import functools

import jax
import jax.numpy as jnp
from jax import lax
from jax.experimental import pallas as pl
from jax.experimental.pallas import tpu as pltpu
from jax.experimental.pallas import tpu_sc as plsc

K = 512
D = 32
BLK = 16384
N = 65536
NB = N // BLK

_NC, _NS = 2, 16
NW = _NC * _NS
BPW = N // NW
CHUNK = 128
NCHUNK = BPW // CHUNK


def _tc_body(x_ref, cb_ref, idx_ref, stats_ref, counts_ref, hcsq_ref):
    step = pl.program_id(0)

    @pl.when(step == 0)
    def _init():
        counts_ref[...] = jnp.zeros_like(counts_ref)
        cb0 = cb_ref[...]
        hcsq_ref[...] = 0.5 * jnp.sum(cb0 * cb0, axis=1, keepdims=True)

    xb = x_ref[...]
    prod = lax.dot_general(cb_ref[...], xb, (((1,), (1,)), ((), ())),
                           preferred_element_type=jnp.float32)
    h = prod - hcsq_ref[...]
    hmax = jnp.max(h, axis=0, keepdims=True)
    mask = h >= hmax
    riota = (K - 1) - lax.broadcasted_iota(jnp.int32, (K, BLK), 0)
    rm = jnp.max(jnp.where(mask, riota, 0), axis=0, keepdims=True)
    idx_ref[0, 0, :] = (K - 1) - rm[0]
    counts_ref[...] += jnp.sum(jnp.where(mask, 1, 0), axis=1, keepdims=True)

    @pl.when(step == NB - 1)
    def _finish():
        p = counts_ref[...].astype(jnp.float32) * (1.0 / N)
        ent = -jnp.sum(p * jnp.log(p + 1e-10))
        perp = jnp.exp(ent)
        div = 1.0 - ent / jnp.log(jnp.float32(K))
        lane = lax.broadcasted_iota(jnp.int32, (1, 128), 1)
        stats_ref[...] = jnp.where(lane == 0, perp,
                                   jnp.where(lane == 1, div, 0.0))


_tc_call = pl.pallas_call(
    _tc_body,
    grid=(NB,),
    in_specs=[
        pl.BlockSpec((BLK, D), lambda i: (i, 0)),
        pl.BlockSpec((K, D), lambda i: (0, 0)),
    ],
    out_specs=[
        pl.BlockSpec((1, 1, BLK), lambda i: (i, 0, 0)),
        pl.BlockSpec((1, 128), lambda i: (0, 0)),
    ],
    out_shape=[
        jax.ShapeDtypeStruct((NB, 1, BLK), jnp.int32),
        jax.ShapeDtypeStruct((1, 128), jnp.float32),
    ],
    scratch_shapes=[pltpu.VMEM((K, 1), jnp.int32),
                    pltpu.VMEM((K, 1), jnp.float32)],
)


@functools.cache
def _make_sc_gather():
    @functools.partial(
        pl.kernel,
        mesh=plsc.VectorSubcoreMesh(core_axis_name="c", subcore_axis_name="s"),
        compiler_params=pltpu.CompilerParams(use_tc_tiling_on_sc=False),
        out_type=jax.ShapeDtypeStruct((N, D), jnp.float32),
        scratch_types=[
            pltpu.VMEM((BPW,), jnp.int32),
            pltpu.VMEM((BPW, D), jnp.float32),
            pltpu.SemaphoreType.DMA,
        ],
    )
    def _sc_gather(cb_hbm, idx_hbm, out_hbm, idx_v, rows_v, sem):
        wid = lax.axis_index("s") * _NC + lax.axis_index("c")
        base = wid * BPW
        pltpu.sync_copy(idx_hbm.at[pl.ds(base, BPW)], idx_v)
        copies = []
        for j in range(NCHUNK):
            copies.append(
                pltpu.async_copy(cb_hbm.at[idx_v.at[pl.ds(j * CHUNK, CHUNK)]],
                                 rows_v.at[pl.ds(j * CHUNK, CHUNK)], sem))
        for c in copies:
            c.wait()
        pltpu.sync_copy(rows_v, out_hbm.at[pl.ds(base, BPW)])

    return _sc_gather


def kernel(x, codebook):
    xr = x.reshape(N, D)
    idx3, stats = _tc_call(xr, codebook)
    idx = idx3.reshape(N)
    quantized = _make_sc_gather()(codebook, idx)
    return quantized, idx, stats[0, 0], stats[0, 1]
